# baseline (speedup 1.0000x reference)
import functools

import jax
import jax.numpy as jnp
from jax import lax
from jax.experimental import pallas as pl
from jax.experimental.pallas import tpu as pltpu
from jax.experimental.pallas import tpu_sc as plsc

NUM_CORES = 2
NUM_SUBCORES = 16
NUM_WORKERS = NUM_CORES * NUM_SUBCORES
CHUNK = 256


def kernel(labels, embedding_table):
    B = labels.shape[0]
    V, D = embedding_table.shape
    b_per_w = B // NUM_WORKERS
    n_chunks = b_per_w // CHUNK

    labels = labels.astype(jnp.int32)
    packed = jnp.concatenate(
        [embedding_table[0 : V - 1 : 2], embedding_table[1 : V - 1 : 2]], axis=1
    )
    pair_idx = labels >> 1
    gidx = (
        (jnp.arange(B, dtype=jnp.int32)[:, None] & (CHUNK - 1)) * (2 * D)
        + (labels[:, None] & 1) * D
        + jnp.arange(D, dtype=jnp.int32)[None, :]
    ).reshape(-1)

    mesh = plsc.VectorSubcoreMesh(core_axis_name="c", subcore_axis_name="s")

    @functools.partial(
        pl.kernel,
        mesh=mesh,
        out_type=jax.ShapeDtypeStruct((B * D,), jnp.float32),
        scratch_types=[
            pltpu.VMEM((CHUNK,), jnp.int32),
            pltpu.VMEM((CHUNK * D,), jnp.int32),
            pltpu.VMEM((CHUNK, 2 * D), jnp.float32),
            pltpu.VMEM((CHUNK * D,), jnp.float32),
            pltpu.SemaphoreType.DMA,
        ],
        compiler_params=pltpu.CompilerParams(needs_layout_passes=False),
    )
    def emb(pidx_hbm, gidx_hbm, table_hbm, out_hbm, pidx_v, gidx_v, prows, orows, sem):
        wid = lax.axis_index("s") * NUM_CORES + lax.axis_index("c")
        for h in range(n_chunks):
            base = wid * b_per_w + h * CHUNK
            pltpu.sync_copy(pidx_hbm.at[pl.ds(base, CHUNK)], pidx_v)
            pltpu.sync_copy(gidx_hbm.at[pl.ds(base * D, CHUNK * D)], gidx_v)
            pltpu.async_copy(table_hbm.at[pidx_v], prows, sem).wait()

            def body(g, carry):
                idx = gidx_v[pl.ds(g * 16, 16)]
                vals = plsc.load_gather(prows, [idx >> 7, idx & 127])
                orows[pl.ds(g * 16, 16)] = vals
                return carry

            lax.fori_loop(0, CHUNK * D // 16, body, 0)
            pltpu.sync_copy(orows, out_hbm.at[pl.ds(base * D, CHUNK * D)])

    out_flat = emb(pair_idx, gidx, packed)
    return out_flat.reshape(B, D)


# speedup vs baseline: 5.5718x; 5.5718x over previous
import functools

import jax
import jax.numpy as jnp
from jax import lax
from jax.experimental import pallas as pl
from jax.experimental.pallas import tpu as pltpu
from jax.experimental.pallas import tpu_sc as plsc

NUM_CORES = 2
NUM_SUBCORES = 16
NUM_WORKERS = NUM_CORES * NUM_SUBCORES
CHUNK = 256
PACK_W = 128


def _pack_kernel(table_t, VP, D):
    n_steps = VP // PACK_W
    mesh = plsc.VectorSubcoreMesh(core_axis_name="c", subcore_axis_name="s")

    @functools.partial(
        pl.kernel,
        mesh=mesh,
        out_type=jax.ShapeDtypeStruct((VP // 2, 2 * D), jnp.float32),
        scratch_types=[
            pltpu.VMEM((2, D, PACK_W), jnp.float32),
            pltpu.VMEM((2, PACK_W // 2, 2 * D), jnp.float32),
            pltpu.SemaphoreType.DMA,
            pltpu.SemaphoreType.DMA,
            pltpu.SemaphoreType.DMA,
            pltpu.SemaphoreType.DMA,
        ],
        compiler_params=pltpu.CompilerParams(needs_layout_passes=False),
    )
    def pack(tab_hbm, out_hbm, inbuf, outbuf, si0, si1, so0, so1):
        wid = lax.axis_index("s") * NUM_CORES + lax.axis_index("c")
        sin = (si0, si1)
        sout = (so0, so1)
        my_steps = (n_steps - wid + NUM_WORKERS - 1) // NUM_WORKERS

        def start_in(j, b):
            v0 = (wid + j * NUM_WORKERS) * PACK_W
            pltpu.make_async_copy(
                tab_hbm.at[:, pl.ds(v0, PACK_W)], inbuf.at[b], sin[b]
            ).start()

        def wait_in(b):
            pltpu.make_async_copy(
                tab_hbm.at[:, pl.ds(0, PACK_W)], inbuf.at[b], sin[b]
            ).wait()

        def transpose_step(b):
            def body(g, carry):
                k = g * 16
                f0 = k & (D - 1)
                u = k >> 7
                h = (k >> 6) & 1
                fidx = f0 + lax.broadcasted_iota(jnp.int32, (16,), 0)
                cidx = jnp.full((16,), 2 * u + h, jnp.int32)
                vals = plsc.load_gather(inbuf.at[b], [fidx, cidx])
                ridx = jnp.full((16,), u, jnp.int32)
                plsc.store_scatter(outbuf.at[b], [ridx, fidx + h * D], vals)
                return carry

            lax.fori_loop(0, PACK_W * D // 16, body, 0, unroll=8)

        def start_out(j, b):
            r0 = (wid + j * NUM_WORKERS) * (PACK_W // 2)
            pltpu.make_async_copy(
                outbuf.at[b], out_hbm.at[pl.ds(r0, PACK_W // 2)], sout[b]
            ).start()

        def wait_out(b):
            pltpu.make_async_copy(
                outbuf.at[b], out_hbm.at[pl.ds(0, PACK_W // 2)], sout[b]
            ).wait()

        @pl.when(my_steps > 0)
        def _():
            start_in(0, 0)

            def loop(p, carry):
                for b in (0, 1):
                    j = 2 * p + b

                    @pl.when(j < my_steps)
                    def _():
                        wait_in(b)

                        @pl.when(j + 1 < my_steps)
                        def _():
                            start_in(j + 1, 1 - b)

                        @pl.when(j >= 2)
                        def _():
                            wait_out(b)

                        transpose_step(b)
                        start_out(j, b)

                return carry

            lax.fori_loop(0, (my_steps + 1) >> 1, loop, 0)
            wait_out(0)
            wait_out(1)

    return pack(table_t)


def kernel(labels, embedding_table):
    B = labels.shape[0]
    V, D = embedding_table.shape
    b_per_w = B // NUM_WORKERS
    n_chunks = b_per_w // CHUNK

    labels = labels.astype(jnp.int32)
    VP = (V - 1) - (V - 1) % (2 * PACK_W)
    packed = _pack_kernel(embedding_table.T, VP, D)
    pair_idx = jnp.minimum(labels, VP - 1) >> 1
    gidx = (
        (jnp.arange(B, dtype=jnp.int32)[:, None] & (CHUNK - 1)) * (2 * D)
        + (labels[:, None] & 1) * D
        + jnp.arange(D, dtype=jnp.int32)[None, :]
    ).reshape(-1)

    mesh = plsc.VectorSubcoreMesh(core_axis_name="c", subcore_axis_name="s")

    @functools.partial(
        pl.kernel,
        mesh=mesh,
        out_type=jax.ShapeDtypeStruct((B * D,), jnp.float32),
        scratch_types=[
            pltpu.VMEM((CHUNK,), jnp.int32),
            pltpu.VMEM((CHUNK * D,), jnp.int32),
            pltpu.VMEM((CHUNK, 2 * D), jnp.float32),
            pltpu.VMEM((CHUNK * D,), jnp.float32),
            pltpu.SemaphoreType.DMA,
        ],
        compiler_params=pltpu.CompilerParams(needs_layout_passes=False),
    )
    def emb(pidx_hbm, gidx_hbm, table_hbm, out_hbm, pidx_v, gidx_v, prows, orows, sem):
        wid = lax.axis_index("s") * NUM_CORES + lax.axis_index("c")
        for h in range(n_chunks):
            base = wid * b_per_w + h * CHUNK
            pltpu.sync_copy(pidx_hbm.at[pl.ds(base, CHUNK)], pidx_v)
            pltpu.sync_copy(gidx_hbm.at[pl.ds(base * D, CHUNK * D)], gidx_v)
            pltpu.async_copy(table_hbm.at[pidx_v], prows, sem).wait()

            def body(g, carry):
                idx = gidx_v[pl.ds(g * 16, 16)]
                vals = plsc.load_gather(prows, [idx >> 7, idx & 127])
                orows[pl.ds(g * 16, 16)] = vals
                return carry

            lax.fori_loop(0, CHUNK * D // 16, body, 0)
            pltpu.sync_copy(orows, out_hbm.at[pl.ds(base * D, CHUNK * D)])

    out_flat = emb(pair_idx, gidx, packed)
    out = out_flat.reshape(B, D)
    tail_vals = jnp.take(
        embedding_table[VP:], jnp.clip(labels - VP, 0, V - 1 - VP), axis=0
    )
    return jnp.where((labels >= VP)[:, None], tail_vals, out)


# speedup vs baseline: 14.5131x; 2.6047x over previous
import functools

import jax
import jax.numpy as jnp
from jax import lax
from jax.experimental import pallas as pl
from jax.experimental.pallas import tpu as pltpu
from jax.experimental.pallas import tpu_sc as plsc

NUM_CORES = 2
NUM_SUBCORES = 16
NUM_WORKERS = NUM_CORES * NUM_SUBCORES


def kernel(labels, embedding_table):
    B = labels.shape[0]
    V, D = embedding_table.shape
    b_per_w = B // NUM_WORKERS

    mesh = plsc.VectorSubcoreMesh(core_axis_name="c", subcore_axis_name="s")

    @functools.partial(
        pl.kernel,
        mesh=mesh,
        out_type=jax.ShapeDtypeStruct((B, D), jnp.float32),
        scratch_types=[
            pltpu.VMEM((b_per_w,), jnp.int32),
            pltpu.SemaphoreType.DMA,
        ],
    )
    def emb(labels_hbm, table_hbm, out_hbm, idx_v, sem):
        wid = lax.axis_index("s") * NUM_CORES + lax.axis_index("c")
        base = wid * b_per_w
        pltpu.sync_copy(labels_hbm.at[pl.ds(base, b_per_w)], idx_v)

        def body(j, carry):
            v = idx_v[pl.ds(j * 16, 16)]
            for k in range(16):
                pltpu.make_async_copy(
                    table_hbm.at[v[k]], out_hbm.at[base + j * 16 + k], sem
                ).start()
            return carry

        lax.fori_loop(0, b_per_w // 16, body, 0)
        pltpu.make_async_copy(
            table_hbm.at[pl.ds(0, b_per_w)],
            out_hbm.at[pl.ds(base, b_per_w)],
            sem,
        ).wait()

    return emb(labels.astype(jnp.int32), embedding_table)
